# baseline (speedup 1.0000x reference)
import jax
import jax.numpy as jnp
from jax.experimental import pallas as pl
from jax.experimental.pallas import tpu as pltpu

HID = 1024
INTER = 512
NE = 8
TB = 512


def _router_kernel(x_ref, gw_ref, logits_ref, rw_ref):
    x = x_ref[...]
    logits = jnp.dot(x, gw_ref[...], preferred_element_type=jnp.float32)
    logits_ref[...] = logits
    m = jnp.max(logits, axis=-1, keepdims=True)
    p = jnp.exp(logits - m)
    rw = p / jnp.sum(p, axis=-1, keepdims=True)
    ids = jax.lax.broadcasted_iota(jnp.int32, rw.shape, 1)
    i1 = jnp.argmax(rw, axis=-1, keepdims=True)
    v1 = jnp.max(rw, axis=-1, keepdims=True)
    masked = jnp.where(ids == i1, -1.0, rw)
    i2 = jnp.argmax(masked, axis=-1, keepdims=True)
    v2 = jnp.max(masked, axis=-1, keepdims=True)
    denom = v1 + v2
    rw_ref[...] = jnp.where(ids == i1, v1 / denom, 0.0) + jnp.where(
        ids == i2, v2 / denom, 0.0
    )


def _moe_kernel(x_ref, rw_ref, gp_ref, up_ref, dp_ref, out_ref):
    e = pl.program_id(0)
    t = pl.program_id(1)
    rows = pl.ds(t * TB, TB)
    x = x_ref[rows, :].astype(jnp.bfloat16)
    g = jnp.dot(x, gp_ref[0].astype(jnp.bfloat16), preferred_element_type=jnp.float32)
    u = jnp.dot(x, up_ref[0].astype(jnp.bfloat16), preferred_element_type=jnp.float32)
    h = ((g * jax.nn.sigmoid(g)) * u).astype(jnp.bfloat16)
    y = jnp.dot(h, dp_ref[0].astype(jnp.bfloat16), preferred_element_type=jnp.float32)
    ids = jax.lax.broadcasted_iota(jnp.int32, (TB, NE), 1)
    w = jnp.sum(jnp.where(ids == e, rw_ref[rows, :], 0.0), axis=1, keepdims=True)

    @pl.when(e == 0)
    def _init():
        out_ref[rows, :] = w * y

    @pl.when(e > 0)
    def _acc():
        out_ref[rows, :] += w * y


def kernel(hidden_states, gate_w, gate_proj_w, up_proj_w, down_proj_w):
    batch, seq_len, dim = hidden_states.shape
    x = hidden_states.reshape(-1, dim)
    T = x.shape[0]

    logits, rw = pl.pallas_call(
        _router_kernel,
        grid=(T // TB,),
        in_specs=[
            pl.BlockSpec((TB, HID), lambda t: (t, 0)),
            pl.BlockSpec((HID, NE), lambda t: (0, 0)),
        ],
        out_specs=[
            pl.BlockSpec((TB, NE), lambda t: (t, 0)),
            pl.BlockSpec((TB, NE), lambda t: (t, 0)),
        ],
        out_shape=[
            jax.ShapeDtypeStruct((T, NE), jnp.float32),
            jax.ShapeDtypeStruct((T, NE), jnp.float32),
        ],
    )(x, gate_w)

    out = pl.pallas_call(
        _moe_kernel,
        grid=(NE, T // TB),
        in_specs=[
            pl.BlockSpec((T, HID), lambda e, t: (0, 0)),
            pl.BlockSpec((T, NE), lambda e, t: (0, 0)),
            pl.BlockSpec((1, HID, INTER), lambda e, t: (e, 0, 0)),
            pl.BlockSpec((1, HID, INTER), lambda e, t: (e, 0, 0)),
            pl.BlockSpec((1, INTER, HID), lambda e, t: (e, 0, 0)),
        ],
        out_specs=pl.BlockSpec((T, HID), lambda e, t: (0, 0)),
        out_shape=jax.ShapeDtypeStruct((T, HID), jnp.float32),
        compiler_params=pltpu.CompilerParams(
            dimension_semantics=("arbitrary", "arbitrary"),
        ),
    )(x, rw, gate_proj_w, up_proj_w, down_proj_w)

    return out.reshape(batch, seq_len, dim), logits


# speedup vs baseline: 1.1453x; 1.1453x over previous
import jax
import jax.numpy as jnp
from jax.experimental import pallas as pl
from jax.experimental.pallas import tpu as pltpu

HID = 1024
INTER = 512
NE = 8
TB = 256


def _router_kernel(x_ref, gw_ref, logits_ref, rw_ref):
    x = x_ref[...]
    logits = jnp.dot(x, gw_ref[...], preferred_element_type=jnp.float32)
    logits_ref[...] = logits
    m = jnp.max(logits, axis=-1, keepdims=True)
    p = jnp.exp(logits - m)
    rw = p / jnp.sum(p, axis=-1, keepdims=True)
    ids = jax.lax.broadcasted_iota(jnp.int32, rw.shape, 1)
    i1 = jnp.argmax(rw, axis=-1, keepdims=True)
    v1 = jnp.max(rw, axis=-1, keepdims=True)
    masked = jnp.where(ids == i1, -1.0, rw)
    i2 = jnp.argmax(masked, axis=-1, keepdims=True)
    v2 = jnp.max(masked, axis=-1, keepdims=True)
    denom = v1 + v2
    rw_ref[...] = jnp.where(ids == i1, v1 / denom, 0.0) + jnp.where(
        ids == i2, v2 / denom, 0.0
    )


def _moe_kernel(x_ref, rw_ref, gp_ref, up_ref, dp_ref, out_ref, h_ref):
    x = x_ref[...]
    rw = rw_ref[...]
    for e in range(NE):
        g = jnp.dot(x, gp_ref[e], preferred_element_type=jnp.float32)
        u = jnp.dot(x, up_ref[e], preferred_element_type=jnp.float32)
        w = rw[:, e : e + 1]
        h_ref[:, e * INTER : (e + 1) * INTER] = (g * jax.nn.sigmoid(g)) * u * w
    out_ref[...] = jnp.dot(h_ref[...], dp_ref[...], preferred_element_type=jnp.float32)


def kernel(hidden_states, gate_w, gate_proj_w, up_proj_w, down_proj_w):
    batch, seq_len, dim = hidden_states.shape
    x = hidden_states.reshape(-1, dim)
    T = x.shape[0]

    logits, rw = pl.pallas_call(
        _router_kernel,
        grid=(T // TB,),
        in_specs=[
            pl.BlockSpec((TB, HID), lambda t: (t, 0)),
            pl.BlockSpec((HID, NE), lambda t: (0, 0)),
        ],
        out_specs=[
            pl.BlockSpec((TB, NE), lambda t: (t, 0)),
            pl.BlockSpec((TB, NE), lambda t: (t, 0)),
        ],
        out_shape=[
            jax.ShapeDtypeStruct((T, NE), jnp.float32),
            jax.ShapeDtypeStruct((T, NE), jnp.float32),
        ],
    )(x, gate_w)

    out = pl.pallas_call(
        _moe_kernel,
        grid=(T // TB,),
        in_specs=[
            pl.BlockSpec((TB, HID), lambda t: (t, 0)),
            pl.BlockSpec((TB, NE), lambda t: (t, 0)),
            pl.BlockSpec((NE, HID, INTER), lambda t: (0, 0, 0)),
            pl.BlockSpec((NE, HID, INTER), lambda t: (0, 0, 0)),
            pl.BlockSpec((NE * INTER, HID), lambda t: (0, 0)),
        ],
        out_specs=pl.BlockSpec((TB, HID), lambda t: (t, 0)),
        out_shape=jax.ShapeDtypeStruct((T, HID), jnp.float32),
        scratch_shapes=[pltpu.VMEM((TB, NE * INTER), jnp.float32)],
        compiler_params=pltpu.CompilerParams(
            dimension_semantics=("arbitrary",),
            vmem_limit_bytes=100 * 1024 * 1024,
        ),
    )(x, rw, gate_proj_w, up_proj_w, down_proj_w.reshape(NE * INTER, HID))

    return out.reshape(batch, seq_len, dim), logits
